# baseline (speedup 1.0000x reference)
import jax
import jax.numpy as jnp
from jax import lax
from jax.experimental import pallas as pl
from jax.experimental.pallas import tpu as pltpu
from jax.experimental.pallas import tpu_sc as plsc

N = 25000
D = 64
E = 800000
NC, NS, L = 2, 16, 16

NPAD = 25088
RP_T = NPAD // NS
ZCH = 16
PAD_IDX = N + 8

K = 128
BK = 6
NCH = 408
NBLK = NCH // BK
EP_T = NCH * K
EPAD = EP_T * NS
NROW = EPAD // K

KA = 128
BKA = 6
NBUFA = 3
GLA = 2
NCHA = EP_T // KA
NBLKA = NCHA // BKA

_mesh = lambda: plsc.VectorSubcoreMesh(core_axis_name="c", subcore_axis_name="s")


def _fill(ref, n, value):
    for i in range(n // L):
        ref[pl.ds(i * L, L)] = jnp.full((L,), value, ref.dtype)


def _hist_body(e3, deg_u, deg_i, i0, i1, ones_v, stage_v, acc, sl0, sl1, ss0, ss1):
    c = lax.axis_index("c")
    s = lax.axis_index("s")
    idxs = [i0, i1]
    sl = [sl0, sl1]
    ss = [ss0, ss1]
    _fill(ones_v, K, 1.0)
    _fill(stage_v, RP_T, 0.0)
    pltpu.sync_copy(stage_v, acc.at[pl.ds(s * RP_T, RP_T)])
    plsc.subcore_barrier()

    def one_dir(row, out_ref):
        def fire_l(blk, p):
            pltpu.async_copy(e3.at[row, pl.ds(s * NCH + blk * BK, BK)], idxs[p], sl[p])

        def wait_l(p):
            pltpu.make_async_copy(e3.at[row, pl.ds(0, BK)], idxs[p], sl[p]).wait()

        def fire_s(p):
            for cc in range(BK):
                pltpu.async_copy(ones_v, acc.at[idxs[p].at[cc]], ss[p], add=True)

        def wait_s(p):
            for cc in range(BK):
                pltpu.make_async_copy(ones_v, acc.at[idxs[p].at[cc]], ss[p]).wait()

        fire_l(0, 0)
        wait_l(0)
        fire_s(0)
        fire_l(1, 1)

        def round_body(r, _):
            for half in range(2):
                blk = 2 * r + 1 + half
                p = (1 + half) % 2
                wait_l(p)
                fire_s(p)
                wait_s(1 - p)
                fire_l(blk + 1, 1 - p)
            return 0

        lax.fori_loop(0, (NBLK - 2) // 2, round_body, 0)
        wait_l(1)
        fire_s(1)
        wait_s(0)
        wait_s(1)
        plsc.subcore_barrier()
        pltpu.sync_copy(acc.at[pl.ds(s * RP_T, RP_T)], stage_v)
        pltpu.sync_copy(stage_v, out_ref.at[pl.ds(s * RP_T, RP_T)])

    @pl.when(c == 0)
    def _():
        one_dir(0, deg_u)

    @pl.when(c == 1)
    def _():
        one_dir(1, deg_i)


def _hist(e3):
    return pl.kernel(
        _hist_body,
        out_type=(
            jax.ShapeDtypeStruct((NPAD,), jnp.float32),
            jax.ShapeDtypeStruct((NPAD,), jnp.float32),
        ),
        mesh=_mesh(),
        scratch_types=[
            pltpu.VMEM((BK, K), jnp.int32),
            pltpu.VMEM((BK, K), jnp.int32),
            pltpu.VMEM((K,), jnp.float32),
            pltpu.VMEM((RP_T,), jnp.float32),
            pltpu.VMEM_SHARED((NPAD,), jnp.float32),
            pltpu.SemaphoreType.DMA,
            pltpu.SemaphoreType.DMA,
            pltpu.SemaphoreType.DMA,
            pltpu.SemaphoreType.DMA,
        ],
        compiler_params=pltpu.CompilerParams(use_tc_tiling_on_sc=False),
    )(e3)


def _scale_body(u_ref, i_ref, du_ref, di_ref, su_ref, si_ref, iu_ref, ii_ref):
    inv_u = lax.rsqrt(jnp.maximum(du_ref[...], 1.0))
    inv_i = lax.rsqrt(jnp.maximum(di_ref[...], 1.0))
    iu_ref[...] = inv_u
    ii_ref[...] = inv_i
    su_ref[...] = u_ref[...] * inv_u
    si_ref[...] = i_ref[...] * inv_i


def _scale(u, i, du, di):
    blk = RP_T
    g = NPAD // blk
    emb = pl.BlockSpec((blk, D), lambda j: (j, 0))
    col = pl.BlockSpec((blk, 1), lambda j: (j, 0))
    return pl.pallas_call(
        _scale_body,
        grid=(g,),
        in_specs=[emb, emb, col, col],
        out_specs=[emb, emb, col, col],
        out_shape=(
            jax.ShapeDtypeStruct((NPAD, D), jnp.float32),
            jax.ShapeDtypeStruct((NPAD, D), jnp.float32),
            jax.ShapeDtypeStruct((NPAD, 1), jnp.float32),
            jax.ShapeDtypeStruct((NPAD, 1), jnp.float32),
        ),
    )(u, i, du, di)


def _agg_body(su, si, e2, zeros2, acc_u, acc_i,
              ig0, ig1, is0, is1, r0, r1, r2, zb0, zb1, acc,
              sg0, sg1, sg2, ssc0, ssc1, ssc2, sl0, sl1, sw0, sw1, szi):
    c = lax.axis_index("c")
    s = lax.axis_index("s")
    idxg = [ig0, ig1]
    idxs = [is0, is1]
    rows = [r0, r1, r2]
    sg = [sg0, sg1, sg2]
    ss = [ssc0, ssc1, ssc2]
    sl = [sl0, sl1]
    zb = [zb0, zb1]
    sw = [sw0, sw1]

    pltpu.sync_copy(zeros2, zb0)
    for z in range(RP_T // ZCH):
        pltpu.async_copy(zb0, acc.at[pl.ds(s * RP_T + z * ZCH, ZCH)], szi)
    for z in range(RP_T // ZCH):
        pltpu.make_async_copy(zb0, acc.at[pl.ds(0, ZCH)], szi).wait()
    plsc.subcore_barrier()

    def one_dir(grow, srow, table, out_ref):
        def fire_l(blk, p):
            base = s * EP_T + blk * (BKA * KA)
            pltpu.async_copy(e2.at[grow, pl.ds(base, BKA * KA)], idxg[p], sl[p])
            pltpu.async_copy(e2.at[srow, pl.ds(base, BKA * KA)], idxs[p], sl[p])

        def wait_l(p):
            pltpu.make_async_copy(e2.at[grow, pl.ds(0, BKA * KA)], idxg[p], sl[p]).wait()
            pltpu.make_async_copy(e2.at[srow, pl.ds(0, BKA * KA)], idxs[p], sl[p]).wait()

        def fire_g(p, cc, q):
            pltpu.async_copy(table.at[idxg[p].at[pl.ds(cc * KA, KA)]], rows[q], sg[q])

        def wait_g(q):
            pltpu.make_async_copy(table.at[idxg[0].at[pl.ds(0, KA)]], rows[q], sg[q]).wait()

        def fire_s(p, rr, q):
            pltpu.async_copy(rows[q], acc.at[idxs[p].at[pl.ds(rr * KA, KA)]], ss[q], add=True)

        def wait_s(q):
            pltpu.make_async_copy(rows[q], acc.at[idxs[0].at[pl.ds(0, KA)]], ss[q]).wait()

        def block_body(blk, p, skip_head, fire_next):
            wait_l(p)
            for cc in range(BKA):
                q = cc % NBUFA
                if not (skip_head and cc < NBUFA):
                    wait_s(q)
                fire_g(p, cc, q)
                if not (skip_head and cc < GLA):
                    q2 = (cc - GLA) % NBUFA
                    pp = p if cc >= GLA else 1 - p
                    rr = (cc - GLA) % BKA
                    wait_g(q2)
                    fire_s(pp, rr, q2)
            if fire_next:
                fire_l(blk + 1, 1 - p)

        fire_l(0, 0)
        block_body(0, 0, True, True)

        def round_body(r, _):
            for half in range(2):
                blk = 2 * r + 1 + half
                block_body(blk, (1 + half) % 2, False, True)
            return 0

        lax.fori_loop(0, (NBLKA - 2) // 2, round_body, 0)
        block_body(NBLKA - 1, (NBLKA - 1) % 2, False, False)

        for jj in range(NCHA - GLA, NCHA):
            q2 = jj % NBUFA
            wait_g(q2)
            fire_s((NBLKA - 1) % 2, jj % BKA, q2)
        for q in range(NBUFA):
            wait_s(q)

        plsc.subcore_barrier()

        for z in range(RP_T // ZCH):
            p = z % 2
            if z >= 2:
                pltpu.make_async_copy(zb[p], out_ref.at[pl.ds(0, ZCH)], sw[p]).wait()
            base = s * RP_T + z * ZCH
            pltpu.sync_copy(acc.at[pl.ds(base, ZCH)], zb[p])
            pltpu.async_copy(zb[p], out_ref.at[pl.ds(base, ZCH)], sw[p])
        for p in range(2):
            pltpu.make_async_copy(zb[p], out_ref.at[pl.ds(0, ZCH)], sw[p]).wait()

    @pl.when(c == 0)
    def _():
        one_dir(0, 1, su, acc_i)

    @pl.when(c == 1)
    def _():
        one_dir(1, 0, si, acc_u)


def _agg(su, si, e2, zeros2):
    return pl.kernel(
        _agg_body,
        out_type=(
            jax.ShapeDtypeStruct((NPAD, D), jnp.float32),
            jax.ShapeDtypeStruct((NPAD, D), jnp.float32),
        ),
        mesh=_mesh(),
        scratch_types=[
            pltpu.VMEM((BKA * KA,), jnp.int32),
            pltpu.VMEM((BKA * KA,), jnp.int32),
            pltpu.VMEM((BKA * KA,), jnp.int32),
            pltpu.VMEM((BKA * KA,), jnp.int32),
            pltpu.VMEM((KA, D), jnp.float32),
            pltpu.VMEM((KA, D), jnp.float32),
            pltpu.VMEM((KA, D), jnp.float32),
            pltpu.VMEM((ZCH, D), jnp.float32),
            pltpu.VMEM((ZCH, D), jnp.float32),
            pltpu.VMEM_SHARED((NPAD, D), jnp.float32),
        ] + [pltpu.SemaphoreType.DMA] * 11,
        compiler_params=pltpu.CompilerParams(use_tc_tiling_on_sc=False),
    )(su, si, e2, zeros2)


def _post_body(au_ref, ai_ref, iu_ref, ii_ref, uo_ref, io_ref):
    uo_ref[...] = au_ref[...] * iu_ref[...]
    io_ref[...] = ai_ref[...] * ii_ref[...]


def _post(au, ai, iu, ii):
    blk = RP_T
    g = NPAD // blk
    emb = pl.BlockSpec((blk, D), lambda j: (j, 0))
    col = pl.BlockSpec((blk, 1), lambda j: (j, 0))
    return pl.pallas_call(
        _post_body,
        grid=(g,),
        in_specs=[emb, emb, col, col],
        out_specs=[emb, emb],
        out_shape=(
            jax.ShapeDtypeStruct((NPAD, D), jnp.float32),
            jax.ShapeDtypeStruct((NPAD, D), jnp.float32),
        ),
    )(au, ai, iu, ii)


@jax.jit
def kernel(user_embed, item_embed, edge_index):
    u = jnp.zeros((NPAD, D), jnp.float32).at[:N].set(user_embed)
    it = jnp.zeros((NPAD, D), jnp.float32).at[:N].set(item_embed)
    e = jnp.full((2, EPAD), PAD_IDX, jnp.int32).at[:, :E].set(edge_index)
    e3 = e.reshape(2, NROW, K)
    zeros2 = jnp.zeros((ZCH, D), jnp.float32)

    deg_u, deg_i = _hist(e3)
    su, si, iu, ii = _scale(u, it, deg_u.reshape(NPAD, 1), deg_i.reshape(NPAD, 1))
    au, ai = _agg(su, si, e, zeros2)
    uo, io = _post(au, ai, iu, ii)
    return uo[:N], io[:N]


# speedup vs baseline: 1.4031x; 1.4031x over previous
import jax
import jax.numpy as jnp
from jax import lax
from jax.experimental import pallas as pl
from jax.experimental.pallas import tpu as pltpu
from jax.experimental.pallas import tpu_sc as plsc

N = 25000
D = 64
E = 800000
NC, NS, L = 2, 16, 16

NPAD = 25088
RP_T = NPAD // NS
ZCH = 56
PAD_IDX = N + 8

K = 128
BK = 8
NCH = 400
NBLK = NCH // BK
EP_T = NCH * K
EPAD = EP_T * NS
NROW = EPAD // K

KA = 128
BKA = 8
NBUFA = 2
GLA = 1
NCHA = EP_T // KA
NBLKA = NCHA // BKA

_mesh = lambda: plsc.VectorSubcoreMesh(core_axis_name="c", subcore_axis_name="s")


def _fill(ref, n, value):
    for i in range(n // L):
        ref[pl.ds(i * L, L)] = jnp.full((L,), value, ref.dtype)


def _hist_body(e3, deg_u, deg_i, i0, i1, ones_v, stage_v, acc, sl0, sl1, ss0, ss1):
    c = lax.axis_index("c")
    s = lax.axis_index("s")
    idxs = [i0, i1]
    sl = [sl0, sl1]
    ss = [ss0, ss1]
    _fill(ones_v, K, 1.0)
    _fill(stage_v, RP_T, 0.0)
    pltpu.sync_copy(stage_v, acc.at[pl.ds(s * RP_T, RP_T)])
    plsc.subcore_barrier()

    def one_dir(row, out_ref):
        def fire_l(blk, p):
            pltpu.async_copy(e3.at[row, pl.ds(s * NCH + blk * BK, BK)], idxs[p], sl[p])

        def wait_l(p):
            pltpu.make_async_copy(e3.at[row, pl.ds(0, BK)], idxs[p], sl[p]).wait()

        def fire_s(p):
            for cc in range(BK):
                pltpu.async_copy(ones_v, acc.at[idxs[p].at[cc]], ss[p], add=True)

        def wait_s(p):
            for cc in range(BK):
                pltpu.make_async_copy(ones_v, acc.at[idxs[p].at[cc]], ss[p]).wait()

        fire_l(0, 0)
        wait_l(0)
        fire_s(0)
        fire_l(1, 1)

        def round_body(r, _):
            for half in range(2):
                blk = 2 * r + 1 + half
                p = (1 + half) % 2
                wait_l(p)
                fire_s(p)
                wait_s(1 - p)
                fire_l(blk + 1, 1 - p)
            return 0

        lax.fori_loop(0, (NBLK - 2) // 2, round_body, 0)
        wait_l(1)
        fire_s(1)
        wait_s(0)
        wait_s(1)
        plsc.subcore_barrier()
        pltpu.sync_copy(acc.at[pl.ds(s * RP_T, RP_T)], stage_v)
        pltpu.sync_copy(stage_v, out_ref.at[pl.ds(s * RP_T, RP_T)])

    @pl.when(c == 0)
    def _():
        one_dir(0, deg_u)

    @pl.when(c == 1)
    def _():
        one_dir(1, deg_i)


def _hist(e3):
    return pl.kernel(
        _hist_body,
        out_type=(
            jax.ShapeDtypeStruct((NPAD,), jnp.float32),
            jax.ShapeDtypeStruct((NPAD,), jnp.float32),
        ),
        mesh=_mesh(),
        scratch_types=[
            pltpu.VMEM((BK, K), jnp.int32),
            pltpu.VMEM((BK, K), jnp.int32),
            pltpu.VMEM((K,), jnp.float32),
            pltpu.VMEM((RP_T,), jnp.float32),
            pltpu.VMEM_SHARED((NPAD,), jnp.float32),
            pltpu.SemaphoreType.DMA,
            pltpu.SemaphoreType.DMA,
            pltpu.SemaphoreType.DMA,
            pltpu.SemaphoreType.DMA,
        ],
        compiler_params=pltpu.CompilerParams(use_tc_tiling_on_sc=False),
    )(e3)


def _scale_body(u_ref, i_ref, du_ref, di_ref, su_ref, si_ref, iu_ref, ii_ref):
    inv_u = lax.rsqrt(jnp.maximum(du_ref[...], 1.0))
    inv_i = lax.rsqrt(jnp.maximum(di_ref[...], 1.0))
    iu_ref[...] = inv_u
    ii_ref[...] = inv_i
    su_ref[...] = u_ref[...] * inv_u
    si_ref[...] = i_ref[...] * inv_i


def _scale(u, i, du, di):
    blk = RP_T
    g = NPAD // blk
    emb = pl.BlockSpec((blk, D), lambda j: (j, 0))
    col = pl.BlockSpec((blk, 1), lambda j: (j, 0))
    return pl.pallas_call(
        _scale_body,
        grid=(g,),
        in_specs=[emb, emb, col, col],
        out_specs=[emb, emb, col, col],
        out_shape=(
            jax.ShapeDtypeStruct((NPAD, D), jnp.float32),
            jax.ShapeDtypeStruct((NPAD, D), jnp.float32),
            jax.ShapeDtypeStruct((NPAD, 1), jnp.float32),
            jax.ShapeDtypeStruct((NPAD, 1), jnp.float32),
        ),
    )(u, i, du, di)


def _agg_body(su, si, e2, zeros2, acc_u, acc_i,
              ig0, ig1, is0, is1, r0, r1, zb0, zb1, acc,
              sg0, sg1, ssc0, ssc1, sl0, sl1, sw0, sw1, szi):
    c = lax.axis_index("c")
    s = lax.axis_index("s")
    idxg = [ig0, ig1]
    idxs = [is0, is1]
    rows = [r0, r1]
    sg = [sg0, sg1]
    ss = [ssc0, ssc1]
    sl = [sl0, sl1]
    zb = [zb0, zb1]
    sw = [sw0, sw1]

    pltpu.sync_copy(zeros2, zb0)
    for z in range(RP_T // ZCH):
        pltpu.async_copy(zb0, acc.at[pl.ds(s * RP_T + z * ZCH, ZCH)], szi)
    for z in range(RP_T // ZCH):
        pltpu.make_async_copy(zb0, acc.at[pl.ds(0, ZCH)], szi).wait()
    plsc.subcore_barrier()

    def one_dir(grow, srow, table, out_ref):
        def fire_l(blk, p):
            base = s * EP_T + blk * (BKA * KA)
            pltpu.async_copy(e2.at[grow, pl.ds(base, BKA * KA)], idxg[p], sl[p])
            pltpu.async_copy(e2.at[srow, pl.ds(base, BKA * KA)], idxs[p], sl[p])

        def wait_l(p):
            pltpu.make_async_copy(e2.at[grow, pl.ds(0, BKA * KA)], idxg[p], sl[p]).wait()
            pltpu.make_async_copy(e2.at[srow, pl.ds(0, BKA * KA)], idxs[p], sl[p]).wait()

        def fire_g(p, cc, q):
            pltpu.async_copy(table.at[idxg[p].at[pl.ds(cc * KA, KA)]], rows[q], sg[q])

        def wait_g(q):
            pltpu.make_async_copy(table.at[idxg[0].at[pl.ds(0, KA)]], rows[q], sg[q]).wait()

        def fire_s(p, rr, q):
            pltpu.async_copy(rows[q], acc.at[idxs[p].at[pl.ds(rr * KA, KA)]], ss[q], add=True)

        def wait_s(q):
            pltpu.make_async_copy(rows[q], acc.at[idxs[0].at[pl.ds(0, KA)]], ss[q]).wait()

        def block_body(blk, p, skip_head, fire_next):
            wait_l(p)
            for cc in range(BKA):
                q = cc % NBUFA
                if not (skip_head and cc < NBUFA):
                    wait_s(q)
                fire_g(p, cc, q)
                if not (skip_head and cc < GLA):
                    q2 = (cc - GLA) % NBUFA
                    pp = p if cc >= GLA else 1 - p
                    rr = (cc - GLA) % BKA
                    wait_g(q2)
                    fire_s(pp, rr, q2)
            if fire_next:
                fire_l(blk + 1, 1 - p)

        fire_l(0, 0)
        block_body(0, 0, True, True)

        def round_body(r, _):
            for half in range(2):
                blk = 2 * r + 1 + half
                block_body(blk, (1 + half) % 2, False, True)
            return 0

        lax.fori_loop(0, (NBLKA - 2) // 2, round_body, 0)
        block_body(NBLKA - 1, (NBLKA - 1) % 2, False, False)

        for jj in range(NCHA - GLA, NCHA):
            q2 = jj % NBUFA
            wait_g(q2)
            fire_s((NBLKA - 1) % 2, jj % BKA, q2)
        for q in range(NBUFA):
            wait_s(q)

        plsc.subcore_barrier()

        for z in range(RP_T // ZCH):
            p = z % 2
            if z >= 2:
                pltpu.make_async_copy(zb[p], out_ref.at[pl.ds(0, ZCH)], sw[p]).wait()
            base = s * RP_T + z * ZCH
            pltpu.sync_copy(acc.at[pl.ds(base, ZCH)], zb[p])
            pltpu.async_copy(zb[p], out_ref.at[pl.ds(base, ZCH)], sw[p])
        for p in range(2):
            pltpu.make_async_copy(zb[p], out_ref.at[pl.ds(0, ZCH)], sw[p]).wait()

    @pl.when(c == 0)
    def _():
        one_dir(0, 1, su, acc_i)

    @pl.when(c == 1)
    def _():
        one_dir(1, 0, si, acc_u)


def _agg(su, si, e2, zeros2):
    return pl.kernel(
        _agg_body,
        out_type=(
            jax.ShapeDtypeStruct((NPAD, D), jnp.float32),
            jax.ShapeDtypeStruct((NPAD, D), jnp.float32),
        ),
        mesh=_mesh(),
        scratch_types=[
            pltpu.VMEM((BKA * KA,), jnp.int32),
            pltpu.VMEM((BKA * KA,), jnp.int32),
            pltpu.VMEM((BKA * KA,), jnp.int32),
            pltpu.VMEM((BKA * KA,), jnp.int32),
            pltpu.VMEM((KA, D), jnp.float32),
            pltpu.VMEM((KA, D), jnp.float32),
            pltpu.VMEM((ZCH, D), jnp.float32),
            pltpu.VMEM((ZCH, D), jnp.float32),
            pltpu.VMEM_SHARED((NPAD, D), jnp.float32),
        ] + [pltpu.SemaphoreType.DMA] * 9,
        compiler_params=pltpu.CompilerParams(use_tc_tiling_on_sc=False),
    )(su, si, e2, zeros2)


def _post_body(au_ref, ai_ref, iu_ref, ii_ref, uo_ref, io_ref):
    uo_ref[...] = au_ref[...] * iu_ref[...]
    io_ref[...] = ai_ref[...] * ii_ref[...]


def _post(au, ai, iu, ii):
    blk = 1000
    g = N // blk
    emb = pl.BlockSpec((blk, D), lambda j: (j, 0))
    col = pl.BlockSpec((blk, 1), lambda j: (j, 0))
    return pl.pallas_call(
        _post_body,
        grid=(g,),
        in_specs=[emb, emb, col, col],
        out_specs=[emb, emb],
        out_shape=(
            jax.ShapeDtypeStruct((N, D), jnp.float32),
            jax.ShapeDtypeStruct((N, D), jnp.float32),
        ),
    )(au, ai, iu, ii)


@jax.jit
def kernel(user_embed, item_embed, edge_index):
    u = jnp.zeros((NPAD, D), jnp.float32).at[:N].set(user_embed)
    it = jnp.zeros((NPAD, D), jnp.float32).at[:N].set(item_embed)
    e = jnp.full((2, EPAD), PAD_IDX, jnp.int32).at[:, :E].set(edge_index)
    e3 = e.reshape(2, NROW, K)
    zeros2 = jnp.zeros((ZCH, D), jnp.float32)

    deg_u, deg_i = _hist(e3)
    su, si, iu, ii = _scale(u, it, deg_u.reshape(NPAD, 1), deg_i.reshape(NPAD, 1))
    au, ai = _agg(su, si, e, zeros2)
    uo, io = _post(au, ai, iu, ii)
    return uo, io
